# baseline (speedup 1.0000x reference)
import functools

import jax
import jax.numpy as jnp
from jax.experimental import pallas as pl


def _gather_kernel(ids_ref, table_ref, out_ref):
    ids = ids_ref[0]
    v = table_ref.shape[0]
    iota = jax.lax.broadcasted_iota(jnp.int32, (ids.shape[0], v), 1)
    onehot = (ids == iota).astype(jnp.float32)
    out_ref[...] = jnp.dot(onehot, table_ref[...],
                           preferred_element_type=jnp.float32)


def _conv_highway_kernel(x_ref, wstack_ref, cb_ref, wp_ref, bp_ref,
                         wg_ref, bg_ref, out_ref, *, kk, tt):
    nb, maxw, e = x_ref.shape
    xm = x_ref[...].reshape(nb * maxw, e)
    acc = jnp.zeros((nb, tt, e), jnp.float32)
    for k in range(kk):
        q = jnp.dot(xm, wstack_ref[k],
                    preferred_element_type=jnp.float32).reshape(nb, maxw, e)
        acc = acc + q[:, k:k + tt, :]
    conv = jnp.maximum(acc + cb_ref[0][None, None, :], 0.0)
    cnn = jnp.max(conv, axis=1)
    proj = jnp.maximum(
        jnp.dot(cnn, wp_ref[...], preferred_element_type=jnp.float32)
        + bp_ref[0][None, :], 0.0)
    gate = jax.nn.sigmoid(
        jnp.dot(cnn, wg_ref[...], preferred_element_type=jnp.float32)
        + bg_ref[0][None, :])
    out_ref[...] = gate * proj + (1.0 - gate) * cnn


def kernel(input_tensor, emb_table, conv_w, conv_b, w_proj, b_proj,
           w_gate, b_gate):
    s, b, maxw = input_tensor.shape
    v, e = emb_table.shape
    kk = conv_w.shape[2]
    n = s * b
    tt = maxw - kk + 1

    ids = input_tensor.astype(jnp.int32).reshape(-1)
    rows = ids.shape[0]
    gblocks = 32
    r = rows // gblocks
    ids3 = ids.reshape(gblocks, r, 1)

    y = pl.pallas_call(
        _gather_kernel,
        grid=(gblocks,),
        in_specs=[
            pl.BlockSpec((1, r, 1), lambda i: (i, 0, 0)),
            pl.BlockSpec((v, e), lambda i: (0, 0)),
        ],
        out_specs=pl.BlockSpec((r, e), lambda i: (i, 0)),
        out_shape=jax.ShapeDtypeStruct((rows, e), jnp.float32),
    )(ids3, emb_table)

    p = y.reshape(n, e, maxw).transpose(0, 2, 1)

    wstack = conv_w.transpose(2, 1, 0)
    cb2 = conv_b.reshape(1, e)
    bp2 = b_proj.reshape(1, e)
    bg2 = b_gate.reshape(1, e)
    wpt = w_proj.T
    wgt = w_gate.T

    nb = 256
    nblocks = n // nb
    out = pl.pallas_call(
        functools.partial(_conv_highway_kernel, kk=kk, tt=tt),
        grid=(nblocks,),
        in_specs=[
            pl.BlockSpec((nb, maxw, e), lambda i: (i, 0, 0)),
            pl.BlockSpec((kk, e, e), lambda i: (0, 0, 0)),
            pl.BlockSpec((1, e), lambda i: (0, 0)),
            pl.BlockSpec((e, e), lambda i: (0, 0)),
            pl.BlockSpec((1, e), lambda i: (0, 0)),
            pl.BlockSpec((e, e), lambda i: (0, 0)),
            pl.BlockSpec((1, e), lambda i: (0, 0)),
        ],
        out_specs=pl.BlockSpec((nb, e), lambda i: (i, 0)),
        out_shape=jax.ShapeDtypeStruct((n, e), jnp.float32),
    )(p, wstack, cb2, wpt, bp2, wgt, bg2)

    return out.reshape(s, b, e)
